# baseline (speedup 1.0000x reference)
import functools

import jax
import jax.numpy as jnp
from jax import lax
from jax.experimental import pallas as pl
from jax.experimental.pallas import tpu as pltpu
from jax.experimental.pallas import tpu_sc as plsc

_CHUNK = 128
_NBUF = 4


@functools.partial(jax.jit, static_argnames=("nc", "ns", "v"))
def _sc_format(pe_t, small2, nc, ns, v):
    d, _ = pe_t.shape
    nw = nc * ns
    n_full = v // 128
    n_groups = n_full + 1
    v128 = n_groups * 128

    mesh = plsc.VectorSubcoreMesh(
        core_axis_name="c", subcore_axis_name="s", num_cores=nc, num_subcores=ns
    )

    @functools.partial(
        pl.kernel,
        out_type=jax.ShapeDtypeStruct((v128 * d,), jnp.float32),
        mesh=mesh,
        scratch_types=[
            pltpu.VMEM((2, d, 128), jnp.float32),
            pltpu.VMEM((2 * 128 * d,), jnp.float32),
            pltpu.VMEM((128 * d,), jnp.float32),
            pltpu.SemaphoreType.DMA((2,)),
            pltpu.SemaphoreType.DMA((2,)),
        ],
        compiler_params=pltpu.CompilerParams(
            use_tc_tiling_on_sc=True,
            needs_layout_passes=False,
            disable_bounds_checks=True,
        ),
    )
    def k(pe_t_hbm, small2_hbm, out_hbm, src_v, dst_v, sm_v, gsem, wsem):
        cid = lax.axis_index("c")
        sid = lax.axis_index("s")
        wid = sid * nc + cid
        n_t = (n_full - 1 - wid) // nw + 1

        base_j = [(lax.iota(jnp.int32, 16) + 16 * j) * d for j in range(8)]

        def load_start(g, b):
            pltpu.async_copy(
                pe_t_hbm.at[:, pl.ds(g * 128, 128)], src_v.at[b], gsem.at[b]
            )

        def load_wait(b):
            pltpu.make_async_copy(
                pe_t_hbm.at[:, pl.ds(0, 128)], src_v.at[b], gsem.at[b]
            ).wait()

        def store_start(g, b):
            pltpu.async_copy(
                dst_v.at[pl.ds(b * 128 * d, 128 * d)],
                out_hbm.at[pl.ds(g * 128 * d, 128 * d)],
                wsem.at[b],
            )

        def store_wait(b):
            pltpu.make_async_copy(
                dst_v.at[pl.ds(0, 128 * d)],
                out_hbm.at[pl.ds(0, 128 * d)],
                wsem.at[b],
            ).wait()

        load_start(wid, 0)

        def outer(t, carry):
            b = t % 2
            g = wid + t * nw

            @pl.when(t + 1 < n_t)
            def _():
                load_start(g + nw, 1 - b)

            load_wait(b)

            @pl.when(t >= 2)
            def _():
                store_wait(b)

            boff = b * 128 * d

            @plsc.parallel_loop(0, d, unroll=8)
            def _(e):
                eb = e + boff
                for j in range(8):
                    vec = src_v[b, e, pl.ds(16 * j, 16)]
                    plsc.store_scatter(dst_v, [base_j[j] + eb], vec)
            store_start(g, b)
            return carry

        lax.fori_loop(0, n_t, outer, 0, unroll=False)

        store_wait(0)
        store_wait(1)

        @pl.when(wid == 0)
        def _():
            pltpu.sync_copy(small2_hbm, sm_v)
            pltpu.sync_copy(sm_v, out_hbm.at[pl.ds(n_full * 128 * d, 128 * d)])

    return k(pe_t, small2)


@functools.partial(jax.jit, static_argnames=("nc", "ns"))
def _sc_gather(ids_2d, ptab, nc, ns):
    n_chunks_total, chunk = ids_2d.shape
    v128, d = ptab.shape
    nw = nc * ns
    n_chunks = n_chunks_total // nw
    n_outer = n_chunks // _NBUF
    assert n_chunks_total == nw * n_outer * _NBUF

    mesh = plsc.VectorSubcoreMesh(
        core_axis_name="c", subcore_axis_name="s", num_cores=nc, num_subcores=ns
    )

    @functools.partial(
        pl.kernel,
        out_type=jax.ShapeDtypeStruct((n_chunks_total * chunk, 128), jnp.float32),
        mesh=mesh,
        scratch_types=[
            pltpu.VMEM((n_chunks, chunk), jnp.int32),
            pltpu.VMEM((_NBUF, chunk, d), jnp.float32),
            pltpu.SemaphoreType.DMA((_NBUF,)),
            pltpu.SemaphoreType.DMA((_NBUF,)),
        ],
        compiler_params=pltpu.CompilerParams(use_tc_tiling_on_sc=False),
    )
    def k(ids_hbm, ptab, out_hbm, idx_v, rows_v, gsem, ssem):
        cid = lax.axis_index("c")
        sid = lax.axis_index("s")
        wid = sid * nc + cid
        cbase = wid * n_chunks

        pltpu.sync_copy(ids_hbm.at[pl.ds(cbase, n_chunks)], idx_v)

        def gather_start(j, b):
            pltpu.async_copy(ptab.at[idx_v.at[j]], rows_v.at[b], gsem.at[b])

        def gather_wait(b):
            pltpu.make_async_copy(
                ptab.at[pl.ds(0, chunk)], rows_v.at[b], gsem.at[b]
            ).wait()

        def scatter_start(j, b):
            pltpu.async_copy(
                rows_v.at[b],
                out_hbm.at[pl.ds((cbase + j) * chunk, chunk), pl.ds(0, d)],
                ssem.at[b],
            )

        def scatter_wait(b):
            pltpu.make_async_copy(
                rows_v.at[b], out_hbm.at[pl.ds(0, chunk), pl.ds(0, d)], ssem.at[b]
            ).wait()

        for b in range(_NBUF):
            gather_start(b, b)

        def outer(g, carry):
            for b in range(_NBUF):
                gather_wait(b)
                scatter_start(g * _NBUF + b, b)
            for b in range(_NBUF):
                scatter_wait(b)
                gather_start((g + 1) * _NBUF + b, b)
            return carry

        lax.fori_loop(0, n_outer - 1, outer, 0, unroll=False)

        g_last = n_outer - 1
        for b in range(_NBUF):
            gather_wait(b)
            scatter_start(g_last * _NBUF + b, b)
        for b in range(_NBUF):
            scatter_wait(b)

    return k(ids_2d, ptab)


def kernel(ids, pe):
    b, s = ids.shape
    v, d = pe.shape
    info = plsc.get_sparse_core_info()
    nc, ns = info.num_cores, info.num_subcores
    ids_2d = ids.reshape(b * s // _CHUNK, _CHUNK).astype(jnp.int32)
    n_full = v // 128
    small2 = jnp.pad(pe[n_full * 128 :], ((0, 128 - (v - n_full * 128)), (0, 0)))
    ptab_flat = _sc_format(jnp.transpose(pe), small2.reshape(128 * d), nc, ns, v)
    rows = _sc_gather(ids_2d, ptab_flat.reshape(-1, d), nc, ns)
    return rows[:, :d].reshape(b, s, d)


# speedup vs baseline: 1.0605x; 1.0605x over previous
import functools

import jax
import jax.numpy as jnp
from jax import lax
from jax.experimental import pallas as pl
from jax.experimental.pallas import tpu as pltpu
from jax.experimental.pallas import tpu_sc as plsc

_CHUNK = 128
_NBUF = 4


@functools.partial(jax.jit, static_argnames=("nc", "ns", "v"))
def _sc_format(pe_t, small2, nc, ns, v):
    d, _ = pe_t.shape
    nw = nc * ns
    n_full = v // 128
    n_groups = n_full + 1
    v128 = n_groups * 128

    mesh = plsc.VectorSubcoreMesh(
        core_axis_name="c", subcore_axis_name="s", num_cores=nc, num_subcores=ns
    )

    @functools.partial(
        pl.kernel,
        out_type=jax.ShapeDtypeStruct((n_groups * 64, 128), jnp.float32),
        mesh=mesh,
        scratch_types=[
            pltpu.VMEM((2 * d, 129), jnp.float32),
            pltpu.VMEM((2 * 64, 128), jnp.float32),
            pltpu.VMEM((64, 128), jnp.float32),
            pltpu.SemaphoreType.DMA((2,)),
            pltpu.SemaphoreType.DMA((2,)),
        ],
        compiler_params=pltpu.CompilerParams(
            use_tc_tiling_on_sc=True,
            needs_layout_passes=False,
            disable_bounds_checks=True,
        ),
    )
    def k(pe_t_hbm, small2_hbm, out_hbm, src_v, dst_v, sm_v, gsem, wsem):
        cid = lax.axis_index("c")
        sid = lax.axis_index("s")
        wid = sid * nc + cid
        n_t = (n_full - 1 - wid) // nw + 1

        erow_j = [lax.iota(jnp.int32, 16) + 16 * j for j in range(d // 16)]

        def load_start(g, b):
            pltpu.async_copy(
                pe_t_hbm.at[:, pl.ds(g * 128, 128)],
                src_v.at[pl.ds(b * d, d), pl.ds(0, 128)],
                gsem.at[b],
            )

        def load_wait(b):
            pltpu.make_async_copy(
                pe_t_hbm.at[:, pl.ds(0, 128)],
                src_v.at[pl.ds(0, d), pl.ds(0, 128)],
                gsem.at[b],
            ).wait()

        def store_start(g, b):
            pltpu.async_copy(
                dst_v.at[pl.ds(b * 64, 64), :],
                out_hbm.at[pl.ds(g * 64, 64), :],
                wsem.at[b],
            )

        def store_wait(b):
            pltpu.make_async_copy(
                dst_v.at[pl.ds(0, 64), :],
                out_hbm.at[pl.ds(0, 64), :],
                wsem.at[b],
            ).wait()

        load_start(wid, 0)

        def outer(t, carry):
            b = t % 2
            g = wid + t * nw

            @pl.when(t + 1 < n_t)
            def _():
                load_start(g + nw, 1 - b)

            load_wait(b)

            @pl.when(t >= 2)
            def _():
                store_wait(b)

            @plsc.parallel_loop(0, 128, unroll=8)
            def _(c):
                csplat = jnp.full((16,), c, jnp.int32)
                drow = b * 64 + (c >> 1)
                dcol = (c & 1) * 64
                for j in range(d // 16):
                    vec = plsc.load_gather(src_v, [erow_j[j] + b * d, csplat])
                    dst_v[drow, pl.ds(dcol + 16 * j, 16)] = vec

            store_start(g, b)
            return carry

        lax.fori_loop(0, n_t, outer, 0, unroll=False)

        store_wait(0)
        store_wait(1)

        @pl.when(wid == 0)
        def _():
            pltpu.sync_copy(small2_hbm, sm_v)
            pltpu.sync_copy(sm_v, out_hbm.at[pl.ds(n_full * 64, 64), :])

    return k(pe_t, small2)


@functools.partial(jax.jit, static_argnames=("nc", "ns"))
def _sc_gather(ids_2d, ptab, nc, ns):
    n_chunks_total, chunk = ids_2d.shape
    v128, d = ptab.shape
    nw = nc * ns
    n_chunks = n_chunks_total // nw
    n_outer = n_chunks // _NBUF
    assert n_chunks_total == nw * n_outer * _NBUF

    mesh = plsc.VectorSubcoreMesh(
        core_axis_name="c", subcore_axis_name="s", num_cores=nc, num_subcores=ns
    )

    @functools.partial(
        pl.kernel,
        out_type=jax.ShapeDtypeStruct((n_chunks_total * chunk, 128), jnp.float32),
        mesh=mesh,
        scratch_types=[
            pltpu.VMEM((n_chunks, chunk), jnp.int32),
            pltpu.VMEM((_NBUF, chunk, d), jnp.float32),
            pltpu.SemaphoreType.DMA((_NBUF,)),
            pltpu.SemaphoreType.DMA((_NBUF,)),
        ],
        compiler_params=pltpu.CompilerParams(use_tc_tiling_on_sc=False),
    )
    def k(ids_hbm, ptab, out_hbm, idx_v, rows_v, gsem, ssem):
        cid = lax.axis_index("c")
        sid = lax.axis_index("s")
        wid = sid * nc + cid
        cbase = wid * n_chunks

        pltpu.sync_copy(ids_hbm.at[pl.ds(cbase, n_chunks)], idx_v)

        def gather_start(j, b):
            pltpu.async_copy(ptab.at[idx_v.at[j]], rows_v.at[b], gsem.at[b])

        def gather_wait(b):
            pltpu.make_async_copy(
                ptab.at[pl.ds(0, chunk)], rows_v.at[b], gsem.at[b]
            ).wait()

        def scatter_start(j, b):
            pltpu.async_copy(
                rows_v.at[b],
                out_hbm.at[pl.ds((cbase + j) * chunk, chunk), pl.ds(0, d)],
                ssem.at[b],
            )

        def scatter_wait(b):
            pltpu.make_async_copy(
                rows_v.at[b], out_hbm.at[pl.ds(0, chunk), pl.ds(0, d)], ssem.at[b]
            ).wait()

        for b in range(_NBUF):
            gather_start(b, b)

        def outer(g, carry):
            for b in range(_NBUF):
                gather_wait(b)
                scatter_start(g * _NBUF + b, b)
            for b in range(_NBUF):
                scatter_wait(b)
                gather_start((g + 1) * _NBUF + b, b)
            return carry

        lax.fori_loop(0, n_outer - 1, outer, 0, unroll=False)

        g_last = n_outer - 1
        for b in range(_NBUF):
            gather_wait(b)
            scatter_start(g_last * _NBUF + b, b)
        for b in range(_NBUF):
            scatter_wait(b)

    return k(ids_2d, ptab)


def kernel(ids, pe):
    b, s = ids.shape
    v, d = pe.shape
    info = plsc.get_sparse_core_info()
    nc, ns = info.num_cores, info.num_subcores
    ids_2d = ids.reshape(b * s // _CHUNK, _CHUNK).astype(jnp.int32)
    n_full = v // 128
    small2 = jnp.pad(pe[n_full * 128 :], ((0, 128 - (v - n_full * 128)), (0, 0)))
    ptab2 = _sc_format(jnp.transpose(pe), small2.reshape(64, 128), nc, ns, v)
    rows = _sc_gather(ids_2d, ptab2.reshape(-1, d), nc, ns)
    return rows[:, :d].reshape(b, s, d)


# speedup vs baseline: 1.2849x; 1.2116x over previous
import functools

import jax
import jax.numpy as jnp
from jax import lax
from jax.experimental import pallas as pl
from jax.experimental.pallas import tpu as pltpu
from jax.experimental.pallas import tpu_sc as plsc

_CHUNK = 128
_NBUF = 4


@functools.partial(jax.jit, static_argnames=("nc", "ns"))
def _sc_gather(ids_2d, ptab, nc, ns):
    n_chunks_total, chunk = ids_2d.shape
    _, d = ptab.shape
    nw = nc * ns
    n_chunks = n_chunks_total // nw
    n_outer = n_chunks // _NBUF
    assert n_chunks_total == nw * n_outer * _NBUF

    mesh = plsc.VectorSubcoreMesh(
        core_axis_name="c", subcore_axis_name="s", num_cores=nc, num_subcores=ns
    )

    @functools.partial(
        pl.kernel,
        out_type=jax.ShapeDtypeStruct((n_chunks_total * chunk, 128), jnp.float32),
        mesh=mesh,
        scratch_types=[
            pltpu.VMEM((n_chunks, chunk), jnp.int32),
            pltpu.VMEM((_NBUF, chunk, d), jnp.float32),
            pltpu.SemaphoreType.DMA((_NBUF,)),
            pltpu.SemaphoreType.DMA((_NBUF,)),
        ],
        compiler_params=pltpu.CompilerParams(use_tc_tiling_on_sc=False),
    )
    def k(ids_hbm, ptab_hbm, out_hbm, idx_v, rows_v, gsem, ssem):
        cid = lax.axis_index("c")
        sid = lax.axis_index("s")
        wid = sid * nc + cid
        cbase = wid * n_chunks

        pltpu.sync_copy(ids_hbm.at[pl.ds(cbase, n_chunks)], idx_v)

        def gather_start(j, b):
            pltpu.async_copy(ptab_hbm.at[idx_v.at[j]], rows_v.at[b], gsem.at[b])

        def gather_wait(b):
            pltpu.make_async_copy(
                ptab_hbm.at[pl.ds(0, chunk)], rows_v.at[b], gsem.at[b]
            ).wait()

        def scatter_start(j, b):
            pltpu.async_copy(
                rows_v.at[b],
                out_hbm.at[pl.ds((cbase + j) * chunk, chunk), pl.ds(0, d)],
                ssem.at[b],
            )

        def scatter_wait(b):
            pltpu.make_async_copy(
                rows_v.at[b], out_hbm.at[pl.ds(0, chunk), pl.ds(0, d)], ssem.at[b]
            ).wait()

        for b in range(_NBUF):
            gather_start(b, b)

        def outer(g, carry):
            for b in range(_NBUF):
                gather_wait(b)
                scatter_start(g * _NBUF + b, b)
            for b in range(_NBUF):
                scatter_wait(b)
                gather_start((g + 1) * _NBUF + b, b)
            return carry

        lax.fori_loop(0, n_outer - 1, outer, 0, unroll=False)

        g_last = n_outer - 1
        for b in range(_NBUF):
            gather_wait(b)
            scatter_start(g_last * _NBUF + b, b)
        for b in range(_NBUF):
            scatter_wait(b)

    return k(ids_2d, ptab)


def kernel(ids, pe):
    b, s = ids.shape
    v, d = pe.shape
    info = plsc.get_sparse_core_info()
    nc, ns = info.num_cores, info.num_subcores
    ids_2d = ids.reshape(b * s // _CHUNK, _CHUNK).astype(jnp.int32)
    rows = _sc_gather(ids_2d, pe, nc, ns)
    return rows[:, :d].reshape(b, s, d)


# speedup vs baseline: 1.2853x; 1.0003x over previous
import functools

import jax
import jax.numpy as jnp
from jax import lax
from jax.experimental import pallas as pl
from jax.experimental.pallas import tpu as pltpu
from jax.experimental.pallas import tpu_sc as plsc

_CHUNK = 128
_NBUF = 8


@functools.partial(jax.jit, static_argnames=("nc", "ns"))
def _sc_gather(ids_2d, ptab, nc, ns):
    n_chunks_total, chunk = ids_2d.shape
    _, d = ptab.shape
    nw = nc * ns
    n_chunks = n_chunks_total // nw
    n_outer = n_chunks // _NBUF
    assert n_chunks_total == nw * n_outer * _NBUF

    mesh = plsc.VectorSubcoreMesh(
        core_axis_name="c", subcore_axis_name="s", num_cores=nc, num_subcores=ns
    )

    @functools.partial(
        pl.kernel,
        out_type=jax.ShapeDtypeStruct((n_chunks_total * chunk, 128), jnp.float32),
        mesh=mesh,
        scratch_types=[
            pltpu.VMEM((n_chunks, chunk), jnp.int32),
            pltpu.VMEM((_NBUF, chunk, d), jnp.float32),
            pltpu.SemaphoreType.DMA((_NBUF,)),
            pltpu.SemaphoreType.DMA((_NBUF,)),
        ],
        compiler_params=pltpu.CompilerParams(use_tc_tiling_on_sc=False),
    )
    def k(ids_hbm, ptab_hbm, out_hbm, idx_v, rows_v, gsem, ssem):
        cid = lax.axis_index("c")
        sid = lax.axis_index("s")
        wid = sid * nc + cid
        cbase = wid * n_chunks

        pltpu.sync_copy(ids_hbm.at[pl.ds(cbase, n_chunks)], idx_v)

        def gather_start(j, b):
            pltpu.async_copy(ptab_hbm.at[idx_v.at[j]], rows_v.at[b], gsem.at[b])

        def gather_wait(b):
            pltpu.make_async_copy(
                ptab_hbm.at[pl.ds(0, chunk)], rows_v.at[b], gsem.at[b]
            ).wait()

        def scatter_start(j, b):
            pltpu.async_copy(
                rows_v.at[b],
                out_hbm.at[pl.ds((cbase + j) * chunk, chunk), pl.ds(0, d)],
                ssem.at[b],
            )

        def scatter_wait(b):
            pltpu.make_async_copy(
                rows_v.at[b], out_hbm.at[pl.ds(0, chunk), pl.ds(0, d)], ssem.at[b]
            ).wait()

        for b in range(_NBUF):
            gather_start(b, b)

        def outer(g, carry):
            for b in range(_NBUF):
                gather_wait(b)
                scatter_start(g * _NBUF + b, b)
            for b in range(_NBUF):
                scatter_wait(b)
                gather_start((g + 1) * _NBUF + b, b)
            return carry

        lax.fori_loop(0, n_outer - 1, outer, 0, unroll=False)

        g_last = n_outer - 1
        for b in range(_NBUF):
            gather_wait(b)
            scatter_start(g_last * _NBUF + b, b)
        for b in range(_NBUF):
            scatter_wait(b)

    return k(ids_2d, ptab)


def kernel(ids, pe):
    b, s = ids.shape
    v, d = pe.shape
    info = plsc.get_sparse_core_info()
    nc, ns = info.num_cores, info.num_subcores
    ids_2d = ids.reshape(b * s // _CHUNK, _CHUNK).astype(jnp.int32)
    rows = _sc_gather(ids_2d, pe, nc, ns)
    return rows[:, :d].reshape(b, s, d)
